# Initial kernel scaffold; baseline (speedup 1.0000x reference)
import jax
import jax.numpy as jnp
from jax.experimental import pallas as pl


def kernel(X_curr, y_prev, edge, mode, ew1, eb1, ew2, eb2, ew3, eb3, nw1, nb1, nw2, nb2, nw3, nb3, dw1, db1, dw2, db2, dw3, db3, dw4, db4):
    raise NotImplementedError("write your pallas kernel here")


# speedup vs baseline: 6.7539x; 6.7539x over previous
import functools

import jax
import jax.numpy as jnp
from jax import lax
from jax.experimental import pallas as pl
from jax.experimental.pallas import tpu as pltpu
from jax.experimental.pallas import tpu_sc as plsc

F32 = jnp.float32

try:
    _info = plsc.get_sparse_core_info()
    NC, NS = _info.num_cores, _info.num_subcores
except Exception:
    NC, NS = 2, 16
NW = NC * NS

B = 128
D = 5
RC = 200


def _gather_body(RW, MC,
                 table, row2d, col2d, srco, dsto,
                 stable, idxr, idxc, bufr, bufc, gsem, wsem):
    cid = lax.axis_index("c")
    sid = lax.axis_index("s")
    wid = sid * NC + cid
    base_row = wid * RW

    @pl.when(sid == 0)
    def _stage():
        pltpu.sync_copy(table, stable)
    plsc.subcore_barrier()

    @pl.loop(0, MC)
    def _chunk(m):
        r0 = base_row + m * RC
        pltpu.sync_copy(row2d.at[pl.ds(r0, RC)], idxr)
        pltpu.sync_copy(col2d.at[pl.ds(r0, RC)], idxc)
        cbase = r0 * B

        @pl.loop(0, RC, step=D)
        def _round(j0):
            not_first = jnp.logical_or(m > 0, j0 > 0)
            for d in range(D):
                @pl.when(not_first)
                def _drain():
                    pltpu.make_async_copy(
                        bufr.at[d], srco.at[pl.ds(cbase, B)], wsem.at[2 * d]).wait()
                    pltpu.make_async_copy(
                        bufc.at[d], dsto.at[pl.ds(cbase, B)], wsem.at[2 * d + 1]).wait()
                pltpu.async_copy(stable.at[idxr.at[j0 + d]], bufr.at[d], gsem.at[2 * d])
                pltpu.async_copy(stable.at[idxc.at[j0 + d]], bufc.at[d], gsem.at[2 * d + 1])
            for d in range(D):
                e0 = cbase + (j0 + d) * B
                pltpu.make_async_copy(
                    stable.at[idxr.at[j0 + d]], bufr.at[d], gsem.at[2 * d]).wait()
                pltpu.async_copy(bufr.at[d], srco.at[pl.ds(e0, B)], wsem.at[2 * d])
                pltpu.make_async_copy(
                    stable.at[idxc.at[j0 + d]], bufc.at[d], gsem.at[2 * d + 1]).wait()
                pltpu.async_copy(bufc.at[d], dsto.at[pl.ds(e0, B)], wsem.at[2 * d + 1])

    for d in range(D):
        pltpu.make_async_copy(bufr.at[d], srco.at[pl.ds(0, B)], wsem.at[2 * d]).wait()
        pltpu.make_async_copy(bufc.at[d], dsto.at[pl.ds(0, B)], wsem.at[2 * d + 1]).wait()


def _sc_gather(table, row2d, col2d, EP):
    NR = EP // B
    RW = NR // NW
    MC = RW // RC
    mesh = plsc.VectorSubcoreMesh(core_axis_name="c", subcore_axis_name="s")
    body = functools.partial(_gather_body, RW, MC)
    return pl.kernel(
        body,
        compiler_params=pltpu.CompilerParams(use_tc_tiling_on_sc=False),
        out_type=[jax.ShapeDtypeStruct((EP, 8), F32),
                  jax.ShapeDtypeStruct((EP, 8), F32)],
        mesh=mesh,
        scratch_types=[
            pltpu.VMEM_SHARED(table.shape, F32),
            pltpu.VMEM((RC, B), jnp.int32),
            pltpu.VMEM((RC, B), jnp.int32),
            pltpu.VMEM((D, B, 8), F32),
            pltpu.VMEM((D, B, 8), F32),
            pltpu.SemaphoreType.DMA((2 * D,)),
            pltpu.SemaphoreType.DMA((2 * D,)),
        ],
    )(table, row2d, col2d)


def _scatter_body(RW, MC, NACC,
                  msgt, col2d, zeros4, parts,
                  idxc, mbuf, ones_v, accum, gsem, asem):
    cid = lax.axis_index("c")
    sid = lax.axis_index("s")
    wid = sid * NC + cid
    base_row = wid * RW

    @pl.when(sid == 0)
    def _init():
        pltpu.sync_copy(zeros4, accum)
    for k in range(B // 16):
        ones_v[pl.ds(16 * k, 16)] = jnp.full((16,), 1.0, F32)
    plsc.subcore_barrier()

    @pl.loop(0, MC)
    def _chunk(m):
        r0 = base_row + m * RC
        pltpu.sync_copy(col2d.at[pl.ds(r0, RC)], idxc)
        cbase = r0 * B

        for c in range(3):
            @pl.loop(0, RC, step=D)
            def _round(j0, c=c):
                not_first = jnp.logical_or(m > 0, jnp.logical_or(j0 > 0, c > 0))
                for d in range(D):
                    @pl.when(not_first)
                    def _drain():
                        pltpu.make_async_copy(
                            mbuf.at[d], accum.at[c].at[idxc.at[0]], asem.at[d]).wait()
                    e0 = cbase + (j0 + d) * B
                    pltpu.async_copy(msgt.at[c].at[pl.ds(e0, B)], mbuf.at[d], gsem.at[d])
                for d in range(D):
                    e0 = cbase + (j0 + d) * B
                    pltpu.make_async_copy(
                        msgt.at[c].at[pl.ds(e0, B)], mbuf.at[d], gsem.at[d]).wait()
                    pltpu.async_copy(mbuf.at[d], accum.at[c].at[idxc.at[j0 + d]],
                                     asem.at[d], add=True)

        @pl.loop(0, RC, step=D)
        def _cnt_round(j0):
            for d in range(D):
                pltpu.make_async_copy(
                    mbuf.at[d], accum.at[3].at[idxc.at[0]], asem.at[d]).wait()
                pltpu.async_copy(ones_v, accum.at[3].at[idxc.at[j0 + d]],
                                 asem.at[d], add=True)

    for d in range(D):
        pltpu.make_async_copy(mbuf.at[d], accum.at[3].at[idxc.at[0]], asem.at[d]).wait()
    plsc.subcore_barrier()

    @pl.when(sid == 0)
    def _dump():
        pltpu.sync_copy(accum, parts.at[cid])


def _sc_scatter(msgt, col2d, zeros4, EP, NACC):
    NR = EP // B
    RW = NR // NW
    MC = RW // RC
    mesh = plsc.VectorSubcoreMesh(core_axis_name="c", subcore_axis_name="s")
    body = functools.partial(_scatter_body, RW, MC, NACC)
    return pl.kernel(
        body,
        compiler_params=pltpu.CompilerParams(use_tc_tiling_on_sc=False),
        out_type=jax.ShapeDtypeStruct((NC, 4, NACC), F32),
        mesh=mesh,
        scratch_types=[
            pltpu.VMEM((RC, B), jnp.int32),
            pltpu.VMEM((D, B), F32),
            pltpu.VMEM((B,), F32),
            pltpu.VMEM_SHARED((4, NACC), F32),
            pltpu.SemaphoreType.DMA((D,)),
            pltpu.SemaphoreType.DMA((D,)),
        ],
    )(msgt, col2d, zeros4)


def _edge_block(src_ref, dst_ref, w1, b1, w2, b2, w3, b3, out_ref):
    s = src_ref[...]
    t = dst_ref[...]
    d = t[:, 0:3] - s[:, 0:3]
    fs = s[:, 4:5]
    fd = t[:, 4:5]
    ea = (fd - fs) * d
    nr = jnp.sqrt(jnp.sum(d * d, axis=1, keepdims=True))
    ni = jnp.concatenate([d, nr, ea, fs, fd], axis=1)
    h = jnp.maximum(jnp.dot(ni, w1[...], preferred_element_type=F32) + b1[...], 0.0)
    h = jnp.maximum(jnp.dot(h, w2[...], preferred_element_type=F32) + b2[...], 0.0)
    o = jnp.dot(h, w3[...], preferred_element_type=F32) + b3[...]
    out_ref[...] = (o + ea).T


def _tc_edge(src8, dst8, ew1, eb1, ew2, eb2, ew3, eb3, EP):
    BE = 6400
    grid = (EP // BE,)
    full = lambda a: pl.BlockSpec(a.shape, lambda i: (0,) * a.ndim)
    return pl.pallas_call(
        _edge_block,
        grid=grid,
        in_specs=[
            pl.BlockSpec((BE, 8), lambda i: (i, 0)),
            pl.BlockSpec((BE, 8), lambda i: (i, 0)),
            full(ew1), full(eb1), full(ew2), full(eb2), full(ew3), full(eb3),
        ],
        out_specs=pl.BlockSpec((3, BE), lambda i: (0, i)),
        out_shape=jax.ShapeDtypeStruct((3, EP), F32),
    )(src8, dst8, ew1, eb1, ew2, eb2, ew3, eb3)


def _node_block(x_ref, p0_ref, p1_ref, y_ref,
                nw1, nb1, nw2, nb2, nw3, nb3,
                dw1, db1, dw2, db2, dw3, db3, dw4, db4, out_ref):
    p = p0_ref[...] + p1_ref[...]
    cnt = jnp.maximum(p[:, 3:4], 1.0)
    aggr = p[:, 0:3] / cnt
    x = x_ref[...]
    ni = jnp.concatenate([x[:, 3:5], aggr], axis=1)
    h = jnp.maximum(jnp.dot(ni, nw1[...], preferred_element_type=F32) + nb1[...], 0.0)
    h = jnp.maximum(jnp.dot(h, nw2[...], preferred_element_type=F32) + nb2[...], 0.0)
    delta = jnp.dot(h, nw3[...], preferred_element_type=F32) + nb3[...]
    x5 = jnp.concatenate([x[:, 0:4], x[:, 4:5] + delta], axis=1)
    h = jnp.maximum(jnp.dot(x5, dw1[...], preferred_element_type=F32) + db1[...], 0.0)
    h = jnp.maximum(jnp.dot(h, dw2[...], preferred_element_type=F32) + db2[...], 0.0)
    h = jnp.maximum(jnp.dot(h, dw3[...], preferred_element_type=F32) + db3[...], 0.0)
    xo = jnp.dot(h, dw4[...], preferred_element_type=F32) + db4[...]
    out_ref[...] = y_ref[...] + xo


def _tc_node(x8, p0, p1, y2d, weights, NNODE):
    BN = 10000
    grid = (NNODE // BN,)
    full = lambda a: pl.BlockSpec(a.shape, lambda i: (0,) * a.ndim)
    return pl.pallas_call(
        _node_block,
        grid=grid,
        in_specs=[
            pl.BlockSpec((BN, 8), lambda i: (i, 0)),
            pl.BlockSpec((BN, 4), lambda i: (i, 0)),
            pl.BlockSpec((BN, 4), lambda i: (i, 0)),
            pl.BlockSpec((BN, 1), lambda i: (i, 0)),
        ] + [full(w) for w in weights],
        out_specs=pl.BlockSpec((BN, 1), lambda i: (i, 0)),
        out_shape=jax.ShapeDtypeStruct((NNODE, 1), F32),
    )(x8, p0, p1, y2d, *weights)


def kernel(X_curr, y_prev, edge, mode, ew1, eb1, ew2, eb2, ew3, eb3,
           nw1, nb1, nw2, nb2, nw3, nb3, dw1, db1, dw2, db2, dw3, db3, dw4, db4):
    NNODE = X_curr.shape[0]
    E = edge.shape[1]

    EPW = B * RC * (-(-(E // NW) // (B * RC)))
    EP = EPW * NW
    NACC = NNODE + 8

    x_in = jnp.concatenate(
        [X_curr[:, 0:3], X_curr[:, 4:5], y_prev[:, None]], axis=1)
    table8 = jnp.pad(x_in, ((0, 8), (0, 3)))

    pad_idx = NNODE + (jnp.arange(EP - E, dtype=jnp.int32) % 8)
    row2d = jnp.concatenate([edge[0].astype(jnp.int32), pad_idx]).reshape(EP // B, B)
    col2d = jnp.concatenate([edge[1].astype(jnp.int32), pad_idx]).reshape(EP // B, B)

    src8, dst8 = _sc_gather(table8, row2d, col2d, EP)

    msgt = _tc_edge(src8, dst8, ew1, eb1.reshape(1, -1), ew2, eb2.reshape(1, -1),
                    ew3, eb3.reshape(1, -1), EP)

    zeros4 = jnp.zeros((4, NACC), F32)
    parts = _sc_scatter(msgt, col2d, zeros4, EP, NACC)

    weights = [nw1, nb1.reshape(1, -1), nw2, nb2.reshape(1, -1),
               nw3, nb3.reshape(1, -1),
               dw1, db1.reshape(1, -1), dw2, db2.reshape(1, -1),
               dw3, db3.reshape(1, -1), dw4, db4.reshape(1, -1)]
    out = _tc_node(table8[:NNODE], parts[0, :, :NNODE].T, parts[1, :, :NNODE].T,
                   y_prev[:, None], weights, NNODE)
    return out[:, 0]
